# baseline (speedup 1.0000x reference)
import jax
import jax.numpy as jnp
from jax.experimental import pallas as pl
from jax.experimental.pallas import tpu as pltpu

_BM = 400


def _support_block(x_ref, wt_ref, s_ref):
    s_ref[...] = jnp.dot(x_ref[...], wt_ref[...], preferred_element_type=jnp.float32)


def _agg_block(s_ref, adj_ref, bias_ref, out_ref):
    out_ref[...] = (
        jnp.dot(adj_ref[...], s_ref[...], preferred_element_type=jnp.float32)
        + bias_ref[...]
    )


def kernel(x, adj, W, bias):
    n, d_in = x.shape
    d_out = W.shape[0]
    wt = W.T
    bias2d = bias.reshape(1, d_out)
    support = pl.pallas_call(
        _support_block,
        out_shape=jax.ShapeDtypeStruct((n, d_out), jnp.float32),
    )(x, wt)
    return pl.pallas_call(
        _agg_block,
        grid=(n // _BM,),
        in_specs=[
            pl.BlockSpec((n, d_out), lambda m: (0, 0)),
            pl.BlockSpec((_BM, n), lambda m: (m, 0)),
            pl.BlockSpec((1, d_out), lambda m: (0, 0)),
        ],
        out_specs=pl.BlockSpec((_BM, d_out), lambda m: (m, 0)),
        out_shape=jax.ShapeDtypeStruct((n, d_out), jnp.float32),
        compiler_params=pltpu.CompilerParams(
            dimension_semantics=("parallel",),
        ),
    )(support, adj, bias2d)
